# baseline (speedup 1.0000x reference)
import functools

import jax
import jax.numpy as jnp
import numpy as np
from jax import lax
from jax.experimental import pallas as pl
from jax.experimental.pallas import tpu as pltpu
from jax.experimental.pallas import tpu_sc as plsc

H = 128
NCHUNK = 4
CL = H // NCHUNK
CW = 3 * CL
NCORE = 2
NSUB = 16
EB = 64
RW = 112


def _xh_body(x_ref, w1_ref, b1_ref, w2_ref, b2_ref, vec_ref, o_ref):
    h = jnp.dot(x_ref[...], w1_ref[...].T, preferred_element_type=jnp.float32)
    h = h + b1_ref[...]
    h = jax.nn.silu(h) * (1.0 / 0.6)
    pad = jnp.zeros((h.shape[0], 2 * H - 2 * CW), jnp.float32)
    for c in range(NCHUNK):
        xh = jnp.dot(h, w2_ref[c].T, preferred_element_type=jnp.float32) + b2_ref[c]
        o_ref[c] = jnp.concatenate([xh, vec_ref[c], pad], axis=1)


def _rbf_body(r_ref, wr_ref, br_ref, ev_ref, o_ref):
    r = r_ref[...]
    evb = ev_ref[...]
    pad = jnp.zeros((r.shape[0], RW - CW - 3), jnp.float32)
    for c in range(NCHUNK):
        d = jnp.dot(r, wr_ref[c].T, preferred_element_type=jnp.float32) + br_ref[c]
        o_ref[c] = jnp.concatenate([d, evb, pad], axis=1)


def _xh_table(x, W1, b1, W2p, b2p, vec_re):
    n = x.shape[0]
    bn = 1000
    return pl.pallas_call(
        _xh_body,
        grid=(n // bn,),
        in_specs=[
            pl.BlockSpec((bn, H), lambda b: (b, 0)),
            pl.BlockSpec((H // 2, H), lambda b: (0, 0)),
            pl.BlockSpec((1, H // 2), lambda b: (0, 0)),
            pl.BlockSpec((NCHUNK, CW, H // 2), lambda b: (0, 0, 0)),
            pl.BlockSpec((NCHUNK, 1, CW), lambda b: (0, 0, 0)),
            pl.BlockSpec((NCHUNK, bn, CW), lambda b: (0, b, 0)),
        ],
        out_specs=pl.BlockSpec((NCHUNK, bn, 2 * H), lambda b: (0, b, 0)),
        out_shape=jax.ShapeDtypeStruct((NCHUNK, n, 2 * H), jnp.float32),
    )(x, W1, b1.reshape(1, -1), W2p, b2p.reshape(NCHUNK, 1, CW), vec_re)


def _rbf_table(rbf, Wrp, brp, ev_pad):
    e = rbf.shape[0]
    be = NSUB * EB
    nr = rbf.shape[1]
    return pl.pallas_call(
        _rbf_body,
        grid=(e // be,),
        in_specs=[
            pl.BlockSpec((be, nr), lambda b: (b, 0)),
            pl.BlockSpec((NCHUNK, CW, nr), lambda b: (0, 0, 0)),
            pl.BlockSpec((NCHUNK, 1, CW), lambda b: (0, 0, 0)),
            pl.BlockSpec((be, 3), lambda b: (b, 0)),
        ],
        out_specs=pl.BlockSpec((NCHUNK, be, RW), lambda b: (0, b, 0)),
        out_shape=jax.ShapeDtypeStruct((NCHUNK, e, RW), jnp.float32),
    )(rbf, Wrp, brp.reshape(NCHUNK, 1, CW), ev_pad)


def _sc_kernel(n, n_pad, e_pad):
    mesh = plsc.VectorSubcoreMesh(core_axis_name="c", subcore_axis_name="s",
                                  num_cores=NCORE, num_subcores=NSUB)
    e_tile = e_pad // NSUB
    nblocks = e_tile // EB
    zr = 8
    nzr = n_pad // NSUB
    rows_out = n_pad // NSUB

    @functools.partial(
        pl.kernel,
        out_type=jax.ShapeDtypeStruct((NCHUNK * n_pad, H), jnp.float32),
        mesh=mesh,
        scratch_types=[
            pltpu.MemorySpace.VMEM_SHARED((n_pad, H), jnp.float32),
            pltpu.MemorySpace.VMEM((2 * EB,), jnp.int32),
            pltpu.MemorySpace.VMEM((2 * EB,), jnp.int32),
            pltpu.MemorySpace.VMEM((EB, RW), jnp.float32),
            pltpu.MemorySpace.VMEM((EB, 2 * H), jnp.float32),
            pltpu.MemorySpace.VMEM((EB, 2 * H), jnp.float32),
            pltpu.MemorySpace.VMEM((EB, H), jnp.float32),
            pltpu.SemaphoreType.DMA,
            pltpu.SemaphoreType.DMA,
        ],
    )
    def k(tbl_hbm, rbf_hbm, ji_hbm, out_hbm,
          acc, jb0, jb1, rbfbuf, gb0, gb1, outbuf, sem0, sem1):
        core = lax.axis_index("c")
        sid = lax.axis_index("s")

        zero16 = jnp.zeros((16,), jnp.float32)

        def chunk_pass(p, _):
            chunk = core * 2 + p
            def zrow(r, _):
                for kk in range(H // 16):
                    outbuf[r, pl.ds(16 * kk, 16)] = zero16
                return 0

            lax.fori_loop(0, zr, zrow, 0)
            zbase = sid * nzr

            def zcopy(b, _):
                pltpu.sync_copy(outbuf.at[pl.ds(0, zr)],
                                acc.at[pl.ds(zbase + b * zr, zr)])
                return 0

            lax.fori_loop(0, nzr // zr, zcopy, 0)
            plsc.subcore_barrier()

            tile_e0 = sid * e_tile
            jbs, gbs, sems = (jb0, jb1), (gb0, gb1), (sem0, sem1)

            for b in range(2):
                pltpu.sync_copy(
                    ji_hbm.at[pl.ds(2 * (chunk * e_pad + tile_e0 + b * EB), 2 * EB)],
                    jbs[b])
                pltpu.async_copy(tbl_hbm.at[jbs[b].at[pl.ds(0, EB)]], gbs[b], sems[b])

            @pl.loop(0, nblocks, step=2)
            def pair(g):
                for b in range(2):
                    blk = g + b
                    e0 = tile_e0 + blk * EB
                    pltpu.sync_copy(rbf_hbm.at[pl.ds(chunk * e_pad + e0, EB)], rbfbuf)
                    gbuf = gbs[b]
                    pltpu.make_async_copy(
                        tbl_hbm.at[jbs[b].at[pl.ds(0, EB)]], gbuf, sems[b]).wait()

                    def group(gg, _):
                        base = gg * 16
                        for lane in range(16):
                            e = base + lane
                            prod = [gbuf[e, pl.ds(16 * kk, 16)] * rbfbuf[e, pl.ds(16 * kk, 16)]
                                    for kk in range(6)]
                            evrow = rbfbuf[e, pl.ds(CW, 16)]
                            for c in range(3):
                                evc = evrow[c]
                                for hh in range(2):
                                    vj = gbuf[e, pl.ds(CW + 32 * c + 16 * hh, 16)]
                                    outbuf[e, pl.ds(32 * c + 16 * hh, 16)] = (
                                        prod[hh] * vj + prod[2 + hh] * evc)
                            outbuf[e, pl.ds(96, 16)] = prod[4]
                            outbuf[e, pl.ds(112, 16)] = prod[5]
                        return 0

                    lax.fori_loop(0, EB // 16, group, 0)
                    pltpu.sync_copy(outbuf, acc.at[jbs[b].at[pl.ds(EB, EB)]], add=True)

                    @pl.when(blk + 2 < nblocks)
                    def _prefetch():
                        pltpu.sync_copy(
                            ji_hbm.at[pl.ds(2 * (chunk * e_pad + e0 + 2 * EB), 2 * EB)],
                            jbs[b])
                        pltpu.async_copy(
                            tbl_hbm.at[jbs[b].at[pl.ds(0, EB)]], gbs[b], sems[b])

            plsc.subcore_barrier()

            r0 = sid * rows_out
            pltpu.sync_copy(acc.at[pl.ds(r0, rows_out)],
                            out_hbm.at[pl.ds(chunk * n_pad + r0, rows_out)])
            plsc.subcore_barrier()
            return 0

        lax.fori_loop(0, NCHUNK // NCORE, chunk_pass, 0)

    return k


def kernel(x, vec, edge_index, edge_rbf, edge_vector, W1, b1, W2, b2, Wr, br):
    n = x.shape[0]
    e = edge_index.shape[1]
    num_rbf = edge_rbf.shape[1]
    inv3 = 1.0 / np.sqrt(3.0)
    invh = 1.0 / np.sqrt(float(H))

    e_pad = ((e + NSUB * EB * 2 - 1) // (NSUB * EB * 2)) * (NSUB * EB * 2)
    n_pad = ((n + NSUB * 8 - 1) // (NSUB * 8)) * (NSUB * 8)

    W2p = W2.reshape(3, NCHUNK, CL, H // 2).transpose(1, 0, 2, 3).reshape(NCHUNK, CW, H // 2)
    b2p = b2.reshape(3, NCHUNK, CL).transpose(1, 0, 2).reshape(NCHUNK, CW)
    Wrp = (Wr * inv3).reshape(3, NCHUNK, CL, num_rbf).transpose(1, 0, 2, 3).reshape(NCHUNK, CW, num_rbf)
    brp = (br * inv3).reshape(3, NCHUNK, CL).transpose(1, 0, 2).reshape(NCHUNK, CW)

    vec_re = (vec * invh).reshape(n, 3, NCHUNK, CL).transpose(2, 0, 1, 3).reshape(NCHUNK, n, CW)
    tbl = _xh_table(x, W1, b1, W2p, b2p, vec_re).reshape(NCHUNK * n, 2 * H)

    rbf_pad = jnp.pad(edge_rbf, ((0, e_pad - e), (0, 0)))
    ev_pad = jnp.pad(edge_vector.astype(jnp.float32) * invh, ((0, e_pad - e), (0, 0)))
    rbf_t = _rbf_table(rbf_pad, Wrp, brp, ev_pad).reshape(NCHUNK * e_pad, RW)

    j32 = jnp.pad(edge_index[0].astype(jnp.int32), (0, e_pad - e))
    i32 = jnp.pad(edge_index[1].astype(jnp.int32), (0, e_pad - e))
    jadj = (j32[None, :] + (jnp.arange(NCHUNK, dtype=jnp.int32) * n)[:, None])
    jadj_b = jadj.reshape(NCHUNK, e_pad // EB, 1, EB)
    i_b = jnp.broadcast_to(i32.reshape(1, e_pad // EB, 1, EB), jadj_b.shape)
    ji = jnp.concatenate([jadj_b, i_b], axis=2).reshape(-1)

    out = _sc_kernel(n, n_pad, e_pad)(tbl, rbf_t, ji)
    out = out.reshape(NCHUNK, n_pad, H)[:, :n]

    d_vec = out[:, :, :CW].reshape(NCHUNK, n, 3, CL).transpose(1, 2, 0, 3).reshape(n, 3, H)
    d_x = out[:, :, CW:].transpose(1, 0, 2).reshape(n, H)
    return (d_x, d_vec)


# speedup vs baseline: 1.0372x; 1.0372x over previous
import functools

import jax
import jax.numpy as jnp
import numpy as np
from jax import lax
from jax.experimental import pallas as pl
from jax.experimental.pallas import tpu as pltpu
from jax.experimental.pallas import tpu_sc as plsc

H = 128
NCHUNK = 4
CL = H // NCHUNK
CW = 3 * CL
NCORE = 2
NSUB = 16
EB = 64
RW = 112


def _xh_body(x_ref, w1_ref, b1_ref, w2_ref, b2_ref, vec_ref, o_ref):
    invh = 1.0 / np.sqrt(float(H))
    h = jnp.dot(x_ref[...], w1_ref[...].T, preferred_element_type=jnp.float32)
    h = h + b1_ref[...]
    h = jax.nn.silu(h) * (1.0 / 0.6)
    vec = vec_ref[...] * invh
    bn = h.shape[0]
    pad = jnp.zeros((bn, 2 * H - 2 * CW), jnp.float32)
    for c in range(NCHUNK):
        xh = jnp.dot(h, w2_ref[c].T, preferred_element_type=jnp.float32) + b2_ref[c]
        vecc = vec[:, :, c * CL:(c + 1) * CL].reshape(bn, CW)
        o_ref[c] = jnp.concatenate([xh, vecc, pad], axis=1)


def _rbf_body(r_ref, wr_ref, br_ref, ev_ref, o_ref):
    invh = 1.0 / np.sqrt(float(H))
    r = r_ref[...]
    evb = ev_ref[...] * invh
    pad = jnp.zeros((r.shape[0], RW - CW - 3), jnp.float32)
    for c in range(NCHUNK):
        d = jnp.dot(r, wr_ref[c].T, preferred_element_type=jnp.float32) + br_ref[c]
        o_ref[c] = jnp.concatenate([d, evb, pad], axis=1)


def _xh_table(x, W1, b1, W2p, b2p, vec):
    n = x.shape[0]
    bn = 1000
    return pl.pallas_call(
        _xh_body,
        grid=(n // bn,),
        in_specs=[
            pl.BlockSpec((bn, H), lambda b: (b, 0)),
            pl.BlockSpec((H // 2, H), lambda b: (0, 0)),
            pl.BlockSpec((1, H // 2), lambda b: (0, 0)),
            pl.BlockSpec((NCHUNK, CW, H // 2), lambda b: (0, 0, 0)),
            pl.BlockSpec((NCHUNK, 1, CW), lambda b: (0, 0, 0)),
            pl.BlockSpec((bn, 3, H), lambda b: (b, 0, 0)),
        ],
        out_specs=pl.BlockSpec((NCHUNK, bn, 2 * H), lambda b: (0, b, 0)),
        out_shape=jax.ShapeDtypeStruct((NCHUNK, n, 2 * H), jnp.float32),
    )(x, W1, b1.reshape(1, -1), W2p, b2p.reshape(NCHUNK, 1, CW), vec)


def _rbf_table(rbf, Wrp, brp, ev, e_pad):
    e = rbf.shape[0]
    be = NSUB * EB
    nr = rbf.shape[1]
    return pl.pallas_call(
        _rbf_body,
        grid=(e // be,),
        in_specs=[
            pl.BlockSpec((be, nr), lambda b: (b, 0)),
            pl.BlockSpec((NCHUNK, CW, nr), lambda b: (0, 0, 0)),
            pl.BlockSpec((NCHUNK, 1, CW), lambda b: (0, 0, 0)),
            pl.BlockSpec((be, 3), lambda b: (b, 0)),
        ],
        out_specs=pl.BlockSpec((NCHUNK, be, RW), lambda b: (0, b, 0)),
        out_shape=jax.ShapeDtypeStruct((NCHUNK, e, RW), jnp.float32),
    )(rbf, Wrp, brp.reshape(NCHUNK, 1, CW), ev)


def _sc_kernel(n, n_pad, e_pad):
    mesh = plsc.VectorSubcoreMesh(core_axis_name="c", subcore_axis_name="s",
                                  num_cores=NCORE, num_subcores=NSUB)
    e_tile = e_pad // NSUB
    nblocks = e_tile // EB
    zr = 8
    nzr = n_pad // NSUB
    rows_out = n_pad // NSUB

    @functools.partial(
        pl.kernel,
        out_type=jax.ShapeDtypeStruct((NCHUNK * n_pad, H), jnp.float32),
        mesh=mesh,
        scratch_types=[
            pltpu.MemorySpace.VMEM_SHARED((n_pad, H), jnp.float32),
            pltpu.MemorySpace.VMEM((2 * EB,), jnp.int32),
            pltpu.MemorySpace.VMEM((2 * EB,), jnp.int32),
            pltpu.MemorySpace.VMEM((EB, RW), jnp.float32),
            pltpu.MemorySpace.VMEM((EB, 2 * H), jnp.float32),
            pltpu.MemorySpace.VMEM((EB, 2 * H), jnp.float32),
            pltpu.MemorySpace.VMEM((EB, H), jnp.float32),
            pltpu.SemaphoreType.DMA,
            pltpu.SemaphoreType.DMA,
        ],
    )
    def k(tbl_hbm, rbf_hbm, ji_hbm, out_hbm,
          acc, jb0, jb1, rbfbuf, gb0, gb1, outbuf, sem0, sem1):
        core = lax.axis_index("c")
        sid = lax.axis_index("s")

        zero16 = jnp.zeros((16,), jnp.float32)

        def chunk_pass(p, _):
            chunk = core * 2 + p
            def zrow(r, _):
                for kk in range(H // 16):
                    outbuf[r, pl.ds(16 * kk, 16)] = zero16
                return 0

            lax.fori_loop(0, zr, zrow, 0)
            zbase = sid * nzr

            def zcopy(b, _):
                pltpu.sync_copy(outbuf.at[pl.ds(0, zr)],
                                acc.at[pl.ds(zbase + b * zr, zr)])
                return 0

            lax.fori_loop(0, nzr // zr, zcopy, 0)
            plsc.subcore_barrier()

            tile_e0 = sid * e_tile
            jbs, gbs, sems = (jb0, jb1), (gb0, gb1), (sem0, sem1)

            for b in range(2):
                pltpu.sync_copy(
                    ji_hbm.at[pl.ds(2 * (chunk * e_pad + tile_e0 + b * EB), 2 * EB)],
                    jbs[b])
                pltpu.async_copy(tbl_hbm.at[jbs[b].at[pl.ds(0, EB)]], gbs[b], sems[b])

            @pl.loop(0, nblocks, step=2)
            def pair(g):
                for b in range(2):
                    blk = g + b
                    e0 = tile_e0 + blk * EB
                    pltpu.sync_copy(rbf_hbm.at[pl.ds(chunk * e_pad + e0, EB)], rbfbuf)
                    gbuf = gbs[b]
                    pltpu.make_async_copy(
                        tbl_hbm.at[jbs[b].at[pl.ds(0, EB)]], gbuf, sems[b]).wait()

                    def group(gg, _):
                        base = gg * 16
                        for lane in range(16):
                            e = base + lane
                            prod = [gbuf[e, pl.ds(16 * kk, 16)] * rbfbuf[e, pl.ds(16 * kk, 16)]
                                    for kk in range(6)]
                            evrow = rbfbuf[e, pl.ds(CW, 16)]
                            for c in range(3):
                                evc = evrow[c]
                                tc0 = prod[2] * evc
                                tc1 = prod[3] * evc
                                for hh, t in ((0, tc0), (1, tc1)):
                                    vj = gbuf[e, pl.ds(CW + 32 * c + 16 * hh, 16)]
                                    outbuf[e, pl.ds(32 * c + 16 * hh, 16)] = (
                                        prod[hh] * vj + t)
                            outbuf[e, pl.ds(96, 16)] = prod[4]
                            outbuf[e, pl.ds(112, 16)] = prod[5]
                        return 0

                    lax.fori_loop(0, EB // 16, group, 0)
                    pltpu.sync_copy(outbuf, acc.at[jbs[b].at[pl.ds(EB, EB)]], add=True)

                    @pl.when(blk + 2 < nblocks)
                    def _prefetch():
                        pltpu.sync_copy(
                            ji_hbm.at[pl.ds(2 * (chunk * e_pad + e0 + 2 * EB), 2 * EB)],
                            jbs[b])
                        pltpu.async_copy(
                            tbl_hbm.at[jbs[b].at[pl.ds(0, EB)]], gbs[b], sems[b])

            plsc.subcore_barrier()

            r0 = sid * rows_out
            pltpu.sync_copy(acc.at[pl.ds(r0, rows_out)],
                            out_hbm.at[pl.ds(chunk * n_pad + r0, rows_out)])
            plsc.subcore_barrier()
            return 0

        lax.fori_loop(0, NCHUNK // NCORE, chunk_pass, 0)

    return k


def kernel(x, vec, edge_index, edge_rbf, edge_vector, W1, b1, W2, b2, Wr, br):
    n = x.shape[0]
    e = edge_index.shape[1]
    num_rbf = edge_rbf.shape[1]
    inv3 = 1.0 / np.sqrt(3.0)
    invh = 1.0 / np.sqrt(float(H))

    e_pad = ((e + NSUB * EB * 2 - 1) // (NSUB * EB * 2)) * (NSUB * EB * 2)
    n_pad = ((n + NSUB * 8 - 1) // (NSUB * 8)) * (NSUB * 8)

    W2p = W2.reshape(3, NCHUNK, CL, H // 2).transpose(1, 0, 2, 3).reshape(NCHUNK, CW, H // 2)
    b2p = b2.reshape(3, NCHUNK, CL).transpose(1, 0, 2).reshape(NCHUNK, CW)
    Wrp = (Wr * inv3).reshape(3, NCHUNK, CL, num_rbf).transpose(1, 0, 2, 3).reshape(NCHUNK, CW, num_rbf)
    brp = (br * inv3).reshape(3, NCHUNK, CL).transpose(1, 0, 2).reshape(NCHUNK, CW)

    tbl = _xh_table(x, W1, b1, W2p, b2p, vec).reshape(NCHUNK * n, 2 * H)
    rbf_pad = jnp.pad(edge_rbf, ((0, e_pad - e), (0, 0)))
    ev_pad = jnp.pad(edge_vector.astype(jnp.float32), ((0, e_pad - e), (0, 0)))
    rbf_t = _rbf_table(rbf_pad, Wrp, brp, ev_pad, e_pad).reshape(NCHUNK * e_pad, RW)

    j32 = jnp.pad(edge_index[0].astype(jnp.int32), (0, e_pad - e))
    i32 = jnp.pad(edge_index[1].astype(jnp.int32), (0, e_pad - e))
    jadj = (j32[None, :] + (jnp.arange(NCHUNK, dtype=jnp.int32) * n)[:, None])
    jadj_b = jadj.reshape(NCHUNK, e_pad // EB, 1, EB)
    i_b = jnp.broadcast_to(i32.reshape(1, e_pad // EB, 1, EB), jadj_b.shape)
    ji = jnp.concatenate([jadj_b, i_b], axis=2).reshape(-1)

    out = _sc_kernel(n, n_pad, e_pad)(tbl, rbf_t, ji)
    out = out.reshape(NCHUNK, n_pad, H)[:, :n]

    d_vec = out[:, :, :CW].reshape(NCHUNK, n, 3, CL).transpose(1, 2, 0, 3).reshape(n, 3, H)
    d_x = out[:, :, CW:].transpose(1, 0, 2).reshape(n, H)
    return (d_x, d_vec)


# speedup vs baseline: 1.0400x; 1.0027x over previous
import functools

import jax
import jax.numpy as jnp
import numpy as np
from jax import lax
from jax.experimental import pallas as pl
from jax.experimental.pallas import tpu as pltpu
from jax.experimental.pallas import tpu_sc as plsc

H = 128
NCHUNK = 4
CL = H // NCHUNK
CW = 3 * CL
NCORE = 2
NSUB = 16
EB = 64
RW = 112


def _xh_body(x_ref, w1_ref, b1_ref, w2_ref, b2_ref, vec_ref, o_ref):
    invh = 1.0 / np.sqrt(float(H))
    h = jnp.dot(x_ref[...], w1_ref[...].T, preferred_element_type=jnp.float32)
    h = h + b1_ref[...]
    h = jax.nn.silu(h) * (1.0 / 0.6)
    vec = vec_ref[...] * invh
    bn = h.shape[0]
    pad = jnp.zeros((bn, 2 * H - 2 * CW), jnp.float32)
    for c in range(NCHUNK):
        xh = jnp.dot(h, w2_ref[c].T, preferred_element_type=jnp.float32) + b2_ref[c]
        vecc = vec[:, :, c * CL:(c + 1) * CL].reshape(bn, CW)
        o_ref[c] = jnp.concatenate([xh, vecc, pad], axis=1)


def _rbf_body(r_ref, wr_ref, br_ref, ev_ref, o_ref):
    invh = 1.0 / np.sqrt(float(H))
    r = r_ref[...]
    evb = ev_ref[...] * invh
    pad = jnp.zeros((r.shape[0], RW - CW - 3), jnp.float32)
    for c in range(NCHUNK):
        d = jnp.dot(r, wr_ref[c].T, preferred_element_type=jnp.float32) + br_ref[c]
        o_ref[c] = jnp.concatenate([d, evb, pad], axis=1)


def _xh_table(x, W1, b1, W2p, b2p, vec):
    n = x.shape[0]
    bn = 1000
    return pl.pallas_call(
        _xh_body,
        grid=(n // bn,),
        in_specs=[
            pl.BlockSpec((bn, H), lambda b: (b, 0)),
            pl.BlockSpec((H // 2, H), lambda b: (0, 0)),
            pl.BlockSpec((1, H // 2), lambda b: (0, 0)),
            pl.BlockSpec((NCHUNK, CW, H // 2), lambda b: (0, 0, 0)),
            pl.BlockSpec((NCHUNK, 1, CW), lambda b: (0, 0, 0)),
            pl.BlockSpec((bn, 3, H), lambda b: (b, 0, 0)),
        ],
        out_specs=pl.BlockSpec((NCHUNK, bn, 2 * H), lambda b: (0, b, 0)),
        out_shape=jax.ShapeDtypeStruct((NCHUNK, n, 2 * H), jnp.float32),
    )(x, W1, b1.reshape(1, -1), W2p, b2p.reshape(NCHUNK, 1, CW), vec)


def _rbf_table(rbf, Wrp, brp, ev, e_pad):
    e = rbf.shape[0]
    be = NSUB * EB
    nr = rbf.shape[1]
    return pl.pallas_call(
        _rbf_body,
        grid=(e // be,),
        in_specs=[
            pl.BlockSpec((be, nr), lambda b: (b, 0)),
            pl.BlockSpec((NCHUNK, CW, nr), lambda b: (0, 0, 0)),
            pl.BlockSpec((NCHUNK, 1, CW), lambda b: (0, 0, 0)),
            pl.BlockSpec((be, 3), lambda b: (b, 0)),
        ],
        out_specs=pl.BlockSpec((NCHUNK, be, RW), lambda b: (0, b, 0)),
        out_shape=jax.ShapeDtypeStruct((NCHUNK, e, RW), jnp.float32),
    )(rbf, Wrp, brp.reshape(NCHUNK, 1, CW), ev)


def _sc_kernel(n, n_pad, e_pad):
    mesh = plsc.VectorSubcoreMesh(core_axis_name="c", subcore_axis_name="s",
                                  num_cores=NCORE, num_subcores=NSUB)
    e_tile = e_pad // NSUB
    nblocks = e_tile // EB
    zr = EB
    nzr = n_pad // NSUB
    rows_out = n_pad // NSUB

    @functools.partial(
        pl.kernel,
        out_type=jax.ShapeDtypeStruct((NCHUNK * n_pad, H), jnp.float32),
        mesh=mesh,
        scratch_types=[
            pltpu.MemorySpace.VMEM_SHARED((n_pad, H), jnp.float32),
            pltpu.MemorySpace.VMEM((2 * EB,), jnp.int32),
            pltpu.MemorySpace.VMEM((2 * EB,), jnp.int32),
            pltpu.MemorySpace.VMEM((EB, RW), jnp.float32),
            pltpu.MemorySpace.VMEM((EB, 2 * H), jnp.float32),
            pltpu.MemorySpace.VMEM((EB, 2 * H), jnp.float32),
            pltpu.MemorySpace.VMEM((EB, H), jnp.float32),
            pltpu.SemaphoreType.DMA,
            pltpu.SemaphoreType.DMA,
        ],
    )
    def k(tbl_hbm, rbf_hbm, ji_hbm, out_hbm,
          acc, jb0, jb1, rbfbuf, gb0, gb1, outbuf, sem0, sem1):
        core = lax.axis_index("c")
        sid = lax.axis_index("s")

        zero16 = jnp.zeros((16,), jnp.float32)

        def chunk_pass(p, _):
            chunk = core * 2 + p
            def zrow(r, _):
                for kk in range(H // 16):
                    outbuf[r, pl.ds(16 * kk, 16)] = zero16
                return 0

            lax.fori_loop(0, zr, zrow, 0)
            zbase = sid * nzr

            def zcopy(b, _):
                pltpu.sync_copy(outbuf.at[pl.ds(0, zr)],
                                acc.at[pl.ds(zbase + b * zr, zr)])
                return 0

            lax.fori_loop(0, nzr // zr, zcopy, 0)
            ztail = nzr % zr
            if ztail:
                pltpu.sync_copy(
                    outbuf.at[pl.ds(0, ztail)],
                    acc.at[pl.ds(zbase + (nzr // zr) * zr, ztail)])
            plsc.subcore_barrier()

            tile_e0 = sid * e_tile
            jbs, gbs, sems = (jb0, jb1), (gb0, gb1), (sem0, sem1)

            for b in range(2):
                pltpu.sync_copy(
                    ji_hbm.at[pl.ds(2 * (chunk * e_pad + tile_e0 + b * EB), 2 * EB)],
                    jbs[b])
                pltpu.async_copy(tbl_hbm.at[jbs[b].at[pl.ds(0, EB)]], gbs[b], sems[b])

            @pl.loop(0, nblocks, step=2)
            def pair(g):
                for b in range(2):
                    blk = g + b
                    e0 = tile_e0 + blk * EB
                    pltpu.sync_copy(rbf_hbm.at[pl.ds(chunk * e_pad + e0, EB)], rbfbuf)
                    gbuf = gbs[b]
                    pltpu.make_async_copy(
                        tbl_hbm.at[jbs[b].at[pl.ds(0, EB)]], gbuf, sems[b]).wait()

                    def group(gg, _):
                        base = gg * 16
                        for lane in range(16):
                            e = base + lane
                            prod = [gbuf[e, pl.ds(16 * kk, 16)] * rbfbuf[e, pl.ds(16 * kk, 16)]
                                    for kk in range(6)]
                            evrow = rbfbuf[e, pl.ds(CW, 16)]
                            for c in range(3):
                                evc = evrow[c]
                                tc0 = prod[2] * evc
                                tc1 = prod[3] * evc
                                for hh, t in ((0, tc0), (1, tc1)):
                                    vj = gbuf[e, pl.ds(CW + 32 * c + 16 * hh, 16)]
                                    outbuf[e, pl.ds(32 * c + 16 * hh, 16)] = (
                                        prod[hh] * vj + t)
                            outbuf[e, pl.ds(96, 16)] = prod[4]
                            outbuf[e, pl.ds(112, 16)] = prod[5]
                        return 0

                    lax.fori_loop(0, EB // 16, group, 0)
                    pltpu.sync_copy(outbuf, acc.at[jbs[b].at[pl.ds(EB, EB)]], add=True)

                    @pl.when(blk + 2 < nblocks)
                    def _prefetch():
                        pltpu.sync_copy(
                            ji_hbm.at[pl.ds(2 * (chunk * e_pad + e0 + 2 * EB), 2 * EB)],
                            jbs[b])
                        pltpu.async_copy(
                            tbl_hbm.at[jbs[b].at[pl.ds(0, EB)]], gbs[b], sems[b])

            plsc.subcore_barrier()

            r0 = sid * rows_out
            pltpu.sync_copy(acc.at[pl.ds(r0, rows_out)],
                            out_hbm.at[pl.ds(chunk * n_pad + r0, rows_out)])
            plsc.subcore_barrier()
            return 0

        lax.fori_loop(0, NCHUNK // NCORE, chunk_pass, 0)

    return k


def kernel(x, vec, edge_index, edge_rbf, edge_vector, W1, b1, W2, b2, Wr, br):
    n = x.shape[0]
    e = edge_index.shape[1]
    num_rbf = edge_rbf.shape[1]
    inv3 = 1.0 / np.sqrt(3.0)
    invh = 1.0 / np.sqrt(float(H))

    e_pad = ((e + NSUB * EB * 2 - 1) // (NSUB * EB * 2)) * (NSUB * EB * 2)
    n_pad = ((n + NSUB * 8 - 1) // (NSUB * 8)) * (NSUB * 8)

    W2p = W2.reshape(3, NCHUNK, CL, H // 2).transpose(1, 0, 2, 3).reshape(NCHUNK, CW, H // 2)
    b2p = b2.reshape(3, NCHUNK, CL).transpose(1, 0, 2).reshape(NCHUNK, CW)
    Wrp = (Wr * inv3).reshape(3, NCHUNK, CL, num_rbf).transpose(1, 0, 2, 3).reshape(NCHUNK, CW, num_rbf)
    brp = (br * inv3).reshape(3, NCHUNK, CL).transpose(1, 0, 2).reshape(NCHUNK, CW)

    tbl = _xh_table(x, W1, b1, W2p, b2p, vec).reshape(NCHUNK * n, 2 * H)
    rbf_pad = jnp.pad(edge_rbf, ((0, e_pad - e), (0, 0)))
    ev_pad = jnp.pad(edge_vector.astype(jnp.float32), ((0, e_pad - e), (0, 0)))
    rbf_t = _rbf_table(rbf_pad, Wrp, brp, ev_pad, e_pad).reshape(NCHUNK * e_pad, RW)

    j32 = jnp.pad(edge_index[0].astype(jnp.int32), (0, e_pad - e))
    i32 = jnp.pad(edge_index[1].astype(jnp.int32), (0, e_pad - e))
    jadj = (j32[None, :] + (jnp.arange(NCHUNK, dtype=jnp.int32) * n)[:, None])
    jadj_b = jadj.reshape(NCHUNK, e_pad // EB, 1, EB)
    i_b = jnp.broadcast_to(i32.reshape(1, e_pad // EB, 1, EB), jadj_b.shape)
    ji = jnp.concatenate([jadj_b, i_b], axis=2).reshape(-1)

    out = _sc_kernel(n, n_pad, e_pad)(tbl, rbf_t, ji)
    out = out.reshape(NCHUNK, n_pad, H)[:, :n]

    d_vec = out[:, :, :CW].reshape(NCHUNK, n, 3, CL).transpose(1, 2, 0, 3).reshape(n, 3, H)
    d_x = out[:, :, CW:].transpose(1, 0, 2).reshape(n, H)
    return (d_x, d_vec)


# speedup vs baseline: 1.0636x; 1.0227x over previous
import functools

import jax
import jax.numpy as jnp
import numpy as np
from jax import lax
from jax.experimental import pallas as pl
from jax.experimental.pallas import tpu as pltpu
from jax.experimental.pallas import tpu_sc as plsc

H = 128
NCHUNK = 4
CL = H // NCHUNK
CW = 3 * CL
NCORE = 2
NSUB = 16
EB = 64
RW = 112


def _xh_body(x_ref, w1_ref, b1_ref, w2_ref, b2_ref, vec_ref, o_ref):
    invh = 1.0 / np.sqrt(float(H))
    h = jnp.dot(x_ref[...], w1_ref[...].T, preferred_element_type=jnp.float32)
    h = h + b1_ref[...]
    h = jax.nn.silu(h) * (1.0 / 0.6)
    vec = vec_ref[...] * invh
    bn = h.shape[0]
    pad = jnp.zeros((bn, 2 * H - 2 * CW), jnp.float32)
    for c in range(NCHUNK):
        xh = jnp.dot(h, w2_ref[c].T, preferred_element_type=jnp.float32) + b2_ref[c]
        vecc = vec[:, :, c * CL:(c + 1) * CL].reshape(bn, CW)
        o_ref[c] = jnp.concatenate([xh, vecc, pad], axis=1)


def _rbf_body(r_ref, wr_ref, br_ref, ev_ref, o_ref):
    invh = 1.0 / np.sqrt(float(H))
    r = r_ref[...]
    evb = ev_ref[...] * invh
    pad = jnp.zeros((r.shape[0], RW - CW - 3), jnp.float32)
    for c in range(NCHUNK):
        d = jnp.dot(r, wr_ref[c].T, preferred_element_type=jnp.float32) + br_ref[c]
        o_ref[c] = jnp.concatenate([d, evb, pad], axis=1)


def _xh_table(x, W1, b1, W2p, b2p, vec):
    n = x.shape[0]
    bn = 2000
    return pl.pallas_call(
        _xh_body,
        grid=(n // bn,),
        in_specs=[
            pl.BlockSpec((bn, H), lambda b: (b, 0)),
            pl.BlockSpec((H // 2, H), lambda b: (0, 0)),
            pl.BlockSpec((1, H // 2), lambda b: (0, 0)),
            pl.BlockSpec((NCHUNK, CW, H // 2), lambda b: (0, 0, 0)),
            pl.BlockSpec((NCHUNK, 1, CW), lambda b: (0, 0, 0)),
            pl.BlockSpec((bn, 3, H), lambda b: (b, 0, 0)),
        ],
        out_specs=pl.BlockSpec((NCHUNK, bn, 2 * H), lambda b: (0, b, 0)),
        out_shape=jax.ShapeDtypeStruct((NCHUNK, n, 2 * H), jnp.float32),
    )(x, W1, b1.reshape(1, -1), W2p, b2p.reshape(NCHUNK, 1, CW), vec)


def _rbf_table(rbf, Wrp, brp, ev, e_pad):
    e = rbf.shape[0]
    be = NSUB * EB * 2
    nr = rbf.shape[1]
    return pl.pallas_call(
        _rbf_body,
        grid=(e // be,),
        in_specs=[
            pl.BlockSpec((be, nr), lambda b: (b, 0)),
            pl.BlockSpec((NCHUNK, CW, nr), lambda b: (0, 0, 0)),
            pl.BlockSpec((NCHUNK, 1, CW), lambda b: (0, 0, 0)),
            pl.BlockSpec((be, 3), lambda b: (b, 0)),
        ],
        out_specs=pl.BlockSpec((NCHUNK, be, RW), lambda b: (0, b, 0)),
        out_shape=jax.ShapeDtypeStruct((NCHUNK, e, RW), jnp.float32),
    )(rbf, Wrp, brp.reshape(NCHUNK, 1, CW), ev)


def _sc_kernel(n, n_pad, e_pad):
    mesh = plsc.VectorSubcoreMesh(core_axis_name="c", subcore_axis_name="s",
                                  num_cores=NCORE, num_subcores=NSUB)
    e_tile = e_pad // NSUB
    nblocks = e_tile // EB
    zr = EB
    nzr = n_pad // NSUB
    rows_out = n_pad // NSUB

    @functools.partial(
        pl.kernel,
        out_type=jax.ShapeDtypeStruct((NCHUNK * n_pad, H), jnp.float32),
        mesh=mesh,
        scratch_types=[
            pltpu.MemorySpace.VMEM_SHARED((n_pad, H), jnp.float32),
            pltpu.MemorySpace.VMEM((2 * EB,), jnp.int32),
            pltpu.MemorySpace.VMEM((2 * EB,), jnp.int32),
            pltpu.MemorySpace.VMEM((EB, RW), jnp.float32),
            pltpu.MemorySpace.VMEM((EB, 2 * H), jnp.float32),
            pltpu.MemorySpace.VMEM((EB, 2 * H), jnp.float32),
            pltpu.MemorySpace.VMEM((EB, H), jnp.float32),
            pltpu.SemaphoreType.DMA,
            pltpu.SemaphoreType.DMA,
        ],
    )
    def k(tbl_hbm, rbf_hbm, ji_hbm, out_hbm,
          acc, jb0, jb1, rbfbuf, gb0, gb1, outbuf, sem0, sem1):
        core = lax.axis_index("c")
        sid = lax.axis_index("s")

        zero16 = jnp.zeros((16,), jnp.float32)

        def chunk_pass(p, _):
            chunk = core * 2 + p
            def zrow(r, _):
                for kk in range(H // 16):
                    outbuf[r, pl.ds(16 * kk, 16)] = zero16
                return 0

            lax.fori_loop(0, zr, zrow, 0)
            zbase = sid * nzr

            def zcopy(b, _):
                pltpu.sync_copy(outbuf.at[pl.ds(0, zr)],
                                acc.at[pl.ds(zbase + b * zr, zr)])
                return 0

            lax.fori_loop(0, nzr // zr, zcopy, 0)
            ztail = nzr % zr
            if ztail:
                pltpu.sync_copy(
                    outbuf.at[pl.ds(0, ztail)],
                    acc.at[pl.ds(zbase + (nzr // zr) * zr, ztail)])
            plsc.subcore_barrier()

            tile_e0 = sid * e_tile
            jbs, gbs, sems = (jb0, jb1), (gb0, gb1), (sem0, sem1)

            for b in range(2):
                pltpu.sync_copy(
                    ji_hbm.at[pl.ds(2 * (chunk * e_pad + tile_e0 + b * EB), 2 * EB)],
                    jbs[b])
                pltpu.async_copy(tbl_hbm.at[jbs[b].at[pl.ds(0, EB)]], gbs[b], sems[b])

            @pl.loop(0, nblocks, step=2)
            def pair(g):
                for b in range(2):
                    blk = g + b
                    e0 = tile_e0 + blk * EB
                    pltpu.sync_copy(rbf_hbm.at[pl.ds(chunk * e_pad + e0, EB)], rbfbuf)
                    gbuf = gbs[b]
                    pltpu.make_async_copy(
                        tbl_hbm.at[jbs[b].at[pl.ds(0, EB)]], gbuf, sems[b]).wait()

                    def group(gg, _):
                        base = gg * 16
                        for lane in range(16):
                            e = base + lane
                            prod = [gbuf[e, pl.ds(16 * kk, 16)] * rbfbuf[e, pl.ds(16 * kk, 16)]
                                    for kk in range(6)]
                            evrow = rbfbuf[e, pl.ds(CW, 16)]
                            for c in range(3):
                                evc = evrow[c]
                                tc0 = prod[2] * evc
                                tc1 = prod[3] * evc
                                for hh, t in ((0, tc0), (1, tc1)):
                                    vj = gbuf[e, pl.ds(CW + 32 * c + 16 * hh, 16)]
                                    outbuf[e, pl.ds(32 * c + 16 * hh, 16)] = (
                                        prod[hh] * vj + t)
                            outbuf[e, pl.ds(96, 16)] = prod[4]
                            outbuf[e, pl.ds(112, 16)] = prod[5]
                        return 0

                    lax.fori_loop(0, EB // 16, group, 0)
                    pltpu.sync_copy(outbuf, acc.at[jbs[b].at[pl.ds(EB, EB)]], add=True)

                    @pl.when(blk + 2 < nblocks)
                    def _prefetch():
                        pltpu.sync_copy(
                            ji_hbm.at[pl.ds(2 * (chunk * e_pad + e0 + 2 * EB), 2 * EB)],
                            jbs[b])
                        pltpu.async_copy(
                            tbl_hbm.at[jbs[b].at[pl.ds(0, EB)]], gbs[b], sems[b])

            plsc.subcore_barrier()

            r0 = sid * rows_out
            pltpu.sync_copy(acc.at[pl.ds(r0, rows_out)],
                            out_hbm.at[pl.ds(chunk * n_pad + r0, rows_out)])
            plsc.subcore_barrier()
            return 0

        lax.fori_loop(0, NCHUNK // NCORE, chunk_pass, 0)

    return k


def kernel(x, vec, edge_index, edge_rbf, edge_vector, W1, b1, W2, b2, Wr, br):
    n = x.shape[0]
    e = edge_index.shape[1]
    num_rbf = edge_rbf.shape[1]
    inv3 = 1.0 / np.sqrt(3.0)
    invh = 1.0 / np.sqrt(float(H))

    e_pad = ((e + NSUB * EB * 2 - 1) // (NSUB * EB * 2)) * (NSUB * EB * 2)
    n_pad = ((n + NSUB * 8 - 1) // (NSUB * 8)) * (NSUB * 8)

    W2p = W2.reshape(3, NCHUNK, CL, H // 2).transpose(1, 0, 2, 3).reshape(NCHUNK, CW, H // 2)
    b2p = b2.reshape(3, NCHUNK, CL).transpose(1, 0, 2).reshape(NCHUNK, CW)
    Wrp = (Wr * inv3).reshape(3, NCHUNK, CL, num_rbf).transpose(1, 0, 2, 3).reshape(NCHUNK, CW, num_rbf)
    brp = (br * inv3).reshape(3, NCHUNK, CL).transpose(1, 0, 2).reshape(NCHUNK, CW)

    tbl = _xh_table(x, W1, b1, W2p, b2p, vec).reshape(NCHUNK * n, 2 * H)
    rbf_pad = jnp.pad(edge_rbf, ((0, e_pad - e), (0, 0)))
    ev_pad = jnp.pad(edge_vector.astype(jnp.float32), ((0, e_pad - e), (0, 0)))
    rbf_t = _rbf_table(rbf_pad, Wrp, brp, ev_pad, e_pad).reshape(NCHUNK * e_pad, RW)

    j32 = jnp.pad(edge_index[0].astype(jnp.int32), (0, e_pad - e))
    i32 = jnp.pad(edge_index[1].astype(jnp.int32), (0, e_pad - e))
    jadj = (j32[None, :] + (jnp.arange(NCHUNK, dtype=jnp.int32) * n)[:, None])
    jadj_b = jadj.reshape(NCHUNK, e_pad // EB, 1, EB)
    i_b = jnp.broadcast_to(i32.reshape(1, e_pad // EB, 1, EB), jadj_b.shape)
    ji = jnp.concatenate([jadj_b, i_b], axis=2).reshape(-1)

    out = _sc_kernel(n, n_pad, e_pad)(tbl, rbf_t, ji)
    out = out.reshape(NCHUNK, n_pad, H)[:, :n]

    d_vec = out[:, :, :CW].reshape(NCHUNK, n, 3, CL).transpose(1, 2, 0, 3).reshape(n, 3, H)
    d_x = out[:, :, CW:].transpose(1, 0, 2).reshape(n, H)
    return (d_x, d_vec)
